# baseline (speedup 1.0000x reference)
import functools

import jax
import jax.numpy as jnp
from jax import lax
from jax.experimental import pallas as pl
from jax.experimental.pallas import tpu as pltpu
from jax.experimental.pallas import tpu_sc as plsc

NUM_NODES = 10000
NUM_EDGES = 320000
DIM = 128
NCORES = 2
NSUB = 16
NWORK = NCORES * NSUB
EDGES_PER_WORKER = NUM_EDGES // NWORK
CHUNK = 80
NCHUNKS = EDGES_PER_WORKER // CHUNK
PAD_NODES = 10240
ROWS_PER_SUB = PAD_NODES // NSUB


def _agg_body(with_deg, *refs):
    if with_deg:
        (h_hbm, src_hbm, dst_hbm, zrows_hbm, ones_hbm,
         acc_out, deg_out,
         sidx0, sidx1, didx0, didx1, rows0, rows1, ones_v, acc_sh,
         g0, g1, s0, s1) = refs
    else:
        (h_hbm, src_hbm, dst_hbm, zrows_hbm,
         acc_out,
         sidx0, sidx1, didx0, didx1, rows0, rows1, acc_sh,
         g0, g1, s0, s1) = refs

    cid = lax.axis_index("c")
    sid = lax.axis_index("s")
    wid = cid * NSUB + sid
    base = wid * EDGES_PER_WORKER
    rbase = sid * ROWS_PER_SUB
    rslice = pl.ds(rbase, ROWS_PER_SUB)

    pltpu.sync_copy(zrows_hbm.at[rslice], acc_sh.at[rslice])
    if with_deg:
        pltpu.sync_copy(ones_hbm, ones_v)
    plsc.subcore_barrier()

    def load_idx(c, sref, dref):
        off = base + c * CHUNK
        pltpu.sync_copy(src_hbm.at[pl.ds(off, CHUNK)], sref)
        pltpu.sync_copy(dst_hbm.at[pl.ds(off, CHUNK)], dref)

    load_idx(0, sidx0, didx0)
    pltpu.async_copy(h_hbm.at[sidx0], rows0, g0)
    pltpu.make_async_copy(h_hbm.at[sidx0], rows0, g0).wait()
    pltpu.async_copy(rows0, acc_sh.at[didx0], s0, add=True)
    load_idx(1, sidx1, didx1)
    pltpu.async_copy(h_hbm.at[sidx1], rows1, g1)

    @pl.loop(1, NCHUNKS - 1, step=2)
    def _(i):
        pltpu.make_async_copy(h_hbm.at[sidx1], rows1, g1).wait()
        pltpu.async_copy(rows1, acc_sh.at[didx1], s1, add=True)
        pltpu.make_async_copy(rows0, acc_sh.at[didx0], s0).wait()
        load_idx(i + 1, sidx0, didx0)
        pltpu.async_copy(h_hbm.at[sidx0], rows0, g0)
        pltpu.make_async_copy(h_hbm.at[sidx0], rows0, g0).wait()
        pltpu.async_copy(rows0, acc_sh.at[didx0], s0, add=True)
        pltpu.make_async_copy(rows1, acc_sh.at[didx1], s1).wait()

        @pl.when(i + 2 < NCHUNKS)
        def _():
            load_idx(i + 2, sidx1, didx1)
            pltpu.async_copy(h_hbm.at[sidx1], rows1, g1)

    pltpu.make_async_copy(rows0, acc_sh.at[didx0], s0).wait()

    plsc.subcore_barrier()
    pltpu.sync_copy(acc_sh.at[rslice], acc_out.at[cid, rslice])

    if with_deg:
        pltpu.sync_copy(zrows_hbm.at[rslice], acc_sh.at[rslice])
        pltpu.async_copy(dst_hbm.at[pl.ds(base, CHUNK)], didx0, g0)
        plsc.subcore_barrier()

        pltpu.make_async_copy(dst_hbm.at[pl.ds(base, CHUNK)], didx0, g0).wait()
        pltpu.async_copy(ones_v, acc_sh.at[didx0], s0, add=True)
        pltpu.async_copy(dst_hbm.at[pl.ds(base + CHUNK, CHUNK)], didx1, g1)

        @pl.loop(1, NCHUNKS - 1, step=2)
        def _(i):
            pltpu.make_async_copy(dst_hbm.at[pl.ds(base, CHUNK)],
                                  didx1, g1).wait()
            pltpu.async_copy(ones_v, acc_sh.at[didx1], s1, add=True)
            pltpu.make_async_copy(ones_v, acc_sh.at[didx0], s0).wait()
            pltpu.async_copy(dst_hbm.at[pl.ds(base + (i + 1) * CHUNK, CHUNK)],
                             didx0, g0)
            pltpu.make_async_copy(dst_hbm.at[pl.ds(base, CHUNK)],
                                  didx0, g0).wait()
            pltpu.async_copy(ones_v, acc_sh.at[didx0], s0, add=True)
            pltpu.make_async_copy(ones_v, acc_sh.at[didx1], s1).wait()

            @pl.when(i + 2 < NCHUNKS)
            def _():
                pltpu.async_copy(
                    dst_hbm.at[pl.ds(base + (i + 2) * CHUNK, CHUNK)],
                    didx1, g1)

        pltpu.make_async_copy(ones_v, acc_sh.at[didx0], s0).wait()

        plsc.subcore_barrier()
        pltpu.sync_copy(acc_sh.at[rslice], deg_out.at[cid, rslice])


def _make_agg(with_deg):
    mesh = plsc.VectorSubcoreMesh(core_axis_name="c", subcore_axis_name="s")
    out_type = [jax.ShapeDtypeStruct((NCORES, PAD_NODES, DIM), jnp.float32)]
    scratch = [
        pltpu.VMEM((CHUNK,), jnp.int32),
        pltpu.VMEM((CHUNK,), jnp.int32),
        pltpu.VMEM((CHUNK,), jnp.int32),
        pltpu.VMEM((CHUNK,), jnp.int32),
        pltpu.VMEM((CHUNK, DIM), jnp.float32),
        pltpu.VMEM((CHUNK, DIM), jnp.float32),
        pltpu.VMEM_SHARED((PAD_NODES, DIM), jnp.float32),
        pltpu.SemaphoreType.DMA,
        pltpu.SemaphoreType.DMA,
        pltpu.SemaphoreType.DMA,
        pltpu.SemaphoreType.DMA,
    ]
    if with_deg:
        out_type.append(jax.ShapeDtypeStruct((NCORES, PAD_NODES, DIM),
                                             jnp.float32))
        scratch.insert(6, pltpu.VMEM((CHUNK, DIM), jnp.float32))
    return pl.kernel(functools.partial(_agg_body, with_deg),
                     out_type=out_type, mesh=mesh, scratch_types=scratch)


_agg_with_deg = _make_agg(True)
_agg_no_deg = _make_agg(False)

BN = 1000


def _dense_body(apply_relu, h_ref, a0_ref, a1_ref, d0_ref, d1_ref,
                ws_ref, wn_ref, b_ref, o_ref):
    deg = jnp.maximum(d0_ref[0, :, 0:1] + d1_ref[0, :, 0:1], 1.0)
    agg = (a0_ref[0] + a1_ref[0]) / deg
    out = (jnp.dot(h_ref[...], ws_ref[...],
                   preferred_element_type=jnp.float32,
                   precision=lax.Precision.HIGHEST)
           + jnp.dot(agg, wn_ref[...],
                     preferred_element_type=jnp.float32,
                     precision=lax.Precision.HIGHEST)
           + b_ref[...])
    if apply_relu:
        out = jnp.maximum(out, 0.0)
    o_ref[...] = out


def _dense(h, acc, deg, w_self, w_neigh, b, apply_relu):
    grid = (NUM_NODES // BN,)
    row_spec = pl.BlockSpec((BN, DIM), lambda i: (i, 0))
    part0_spec = pl.BlockSpec((1, BN, DIM), lambda i: (0, i, 0))
    part1_spec = pl.BlockSpec((1, BN, DIM), lambda i: (1, i, 0))
    w_spec = pl.BlockSpec((DIM, DIM), lambda i: (0, 0))
    b_spec = pl.BlockSpec((1, DIM), lambda i: (0, 0))
    return pl.pallas_call(
        functools.partial(_dense_body, apply_relu),
        grid=grid,
        in_specs=[row_spec, part0_spec, part1_spec, part0_spec, part1_spec,
                  w_spec, w_spec, b_spec],
        out_specs=row_spec,
        out_shape=jax.ShapeDtypeStruct((NUM_NODES, DIM), jnp.float32),
    )(h, acc, acc, deg, deg, w_self, w_neigh, b.reshape(1, DIM))


def kernel(x, edge_index, W_self1, W_neigh1, b1, W_self2, W_neigh2, b2):
    src = edge_index[0]
    dst = edge_index[1]
    zrows = jnp.zeros((PAD_NODES, DIM), jnp.float32)
    ones = jnp.ones((CHUNK, DIM), jnp.float32)

    acc1, deg = _agg_with_deg(x, src, dst, zrows, ones)
    h1 = _dense(x, acc1, deg, W_self1, W_neigh1, b1, apply_relu=True)
    (acc2,) = _agg_no_deg(h1, src, dst, zrows)
    h2 = _dense(h1, acc2, deg, W_self2, W_neigh2, b2, apply_relu=False)
    return h2


# speedup vs baseline: 1.2946x; 1.2946x over previous
import functools

import jax
import jax.numpy as jnp
from jax import lax
from jax.experimental import pallas as pl
from jax.experimental.pallas import tpu as pltpu
from jax.experimental.pallas import tpu_sc as plsc

NUM_NODES = 10000
NUM_EDGES = 320000
DIM = 128
NCORES = 2
NSUB = 16
NWORK = NCORES * NSUB
EDGES_PER_WORKER = NUM_EDGES // NWORK
CHUNK = 80
NCHUNKS = EDGES_PER_WORKER // CHUNK
PAD_NODES = 10240
ROWS_PER_SUB = PAD_NODES // NSUB


def _agg_body(with_deg, *refs):
    if with_deg:
        (h_hbm, src_hbm, dst_hbm, zrows_hbm, ones_hbm,
         acc_out, deg_out,
         sidx0, sidx1, didx0, didx1, rows0, rows1, ones_v, acc_sh,
         g0, g1, s0, s1) = refs
    else:
        (h_hbm, src_hbm, dst_hbm, zrows_hbm,
         acc_out,
         sidx0, sidx1, didx0, didx1, rows0, rows1, acc_sh,
         g0, g1, s0, s1) = refs

    cid = lax.axis_index("c")
    sid = lax.axis_index("s")
    wid = cid * NSUB + sid
    base = wid * EDGES_PER_WORKER
    rbase = sid * ROWS_PER_SUB
    rslice = pl.ds(rbase, ROWS_PER_SUB)

    pltpu.sync_copy(zrows_hbm.at[rslice], acc_sh.at[rslice])
    if with_deg:
        pltpu.sync_copy(ones_hbm, ones_v)
    plsc.subcore_barrier()

    def load_idx(c, sref, dref):
        off = base + c * CHUNK
        pltpu.sync_copy(src_hbm.at[pl.ds(off, CHUNK)], sref)
        pltpu.sync_copy(dst_hbm.at[pl.ds(off, CHUNK)], dref)

    load_idx(0, sidx0, didx0)
    pltpu.async_copy(h_hbm.at[sidx0], rows0, g0)

    @pl.loop(0, NCHUNKS - 1, step=2)
    def _(i):
        load_idx(i + 1, sidx1, didx1)
        pltpu.async_copy(h_hbm.at[sidx1], rows1, g1)
        pltpu.make_async_copy(h_hbm.at[sidx0], rows0, g0).wait()
        pltpu.sync_copy(rows0, acc_sh.at[didx0], add=True)
        load_idx(i + 2, sidx0, didx0)
        pltpu.async_copy(h_hbm.at[sidx0], rows0, g0)
        pltpu.make_async_copy(h_hbm.at[sidx1], rows1, g1).wait()
        pltpu.sync_copy(rows1, acc_sh.at[didx1], add=True)

    pltpu.make_async_copy(h_hbm.at[sidx0], rows0, g0).wait()
    pltpu.sync_copy(rows0, acc_sh.at[didx0], add=True)

    plsc.subcore_barrier()
    pltpu.sync_copy(acc_sh.at[rslice], acc_out.at[cid, rslice])

    if with_deg:
        pltpu.sync_copy(zrows_hbm.at[rslice], acc_sh.at[rslice])
        pltpu.async_copy(dst_hbm.at[pl.ds(base, CHUNK)], didx0, g0)
        plsc.subcore_barrier()

        @pl.loop(0, NCHUNKS - 1, step=2)
        def _(i):
            pltpu.async_copy(dst_hbm.at[pl.ds(base + (i + 1) * CHUNK, CHUNK)],
                             didx1, g1)
            pltpu.make_async_copy(dst_hbm.at[pl.ds(base, CHUNK)],
                                  didx0, g0).wait()
            pltpu.sync_copy(ones_v, acc_sh.at[didx0], add=True)
            pltpu.async_copy(dst_hbm.at[pl.ds(base + (i + 2) * CHUNK, CHUNK)],
                             didx0, g0)
            pltpu.make_async_copy(dst_hbm.at[pl.ds(base, CHUNK)],
                                  didx1, g1).wait()
            pltpu.sync_copy(ones_v, acc_sh.at[didx1], add=True)

        pltpu.make_async_copy(dst_hbm.at[pl.ds(base, CHUNK)], didx0, g0).wait()
        pltpu.sync_copy(ones_v, acc_sh.at[didx0], add=True)

        plsc.subcore_barrier()
        pltpu.sync_copy(acc_sh.at[rslice], deg_out.at[cid, rslice])


def _make_agg(with_deg):
    mesh = plsc.VectorSubcoreMesh(core_axis_name="c", subcore_axis_name="s")
    out_type = [jax.ShapeDtypeStruct((NCORES, PAD_NODES, DIM), jnp.float32)]
    scratch = [
        pltpu.VMEM((CHUNK,), jnp.int32),
        pltpu.VMEM((CHUNK,), jnp.int32),
        pltpu.VMEM((CHUNK,), jnp.int32),
        pltpu.VMEM((CHUNK,), jnp.int32),
        pltpu.VMEM((CHUNK, DIM), jnp.float32),
        pltpu.VMEM((CHUNK, DIM), jnp.float32),
        pltpu.VMEM_SHARED((PAD_NODES, DIM), jnp.float32),
        pltpu.SemaphoreType.DMA,
        pltpu.SemaphoreType.DMA,
        pltpu.SemaphoreType.DMA,
        pltpu.SemaphoreType.DMA,
    ]
    if with_deg:
        out_type.append(jax.ShapeDtypeStruct((NCORES, PAD_NODES, DIM),
                                             jnp.float32))
        scratch.insert(6, pltpu.VMEM((CHUNK, DIM), jnp.float32))
    return pl.kernel(functools.partial(_agg_body, with_deg),
                     out_type=out_type, mesh=mesh, scratch_types=scratch)


_agg_with_deg = _make_agg(True)
_agg_no_deg = _make_agg(False)

BN = 1000


def _dense_body(apply_relu, h_ref, a0_ref, a1_ref, d0_ref, d1_ref,
                ws_ref, wn_ref, b_ref, o_ref):
    deg = jnp.maximum(d0_ref[0, :, 0:1] + d1_ref[0, :, 0:1], 1.0)
    agg = (a0_ref[0] + a1_ref[0]) / deg
    out = (jnp.dot(h_ref[...], ws_ref[...],
                   preferred_element_type=jnp.float32,
                   precision=lax.Precision.HIGHEST)
           + jnp.dot(agg, wn_ref[...],
                     preferred_element_type=jnp.float32,
                     precision=lax.Precision.HIGHEST)
           + b_ref[...])
    if apply_relu:
        out = jnp.maximum(out, 0.0)
    o_ref[...] = out


def _dense(h, acc, deg, w_self, w_neigh, b, apply_relu):
    grid = (NUM_NODES // BN,)
    row_spec = pl.BlockSpec((BN, DIM), lambda i: (i, 0))
    part0_spec = pl.BlockSpec((1, BN, DIM), lambda i: (0, i, 0))
    part1_spec = pl.BlockSpec((1, BN, DIM), lambda i: (1, i, 0))
    w_spec = pl.BlockSpec((DIM, DIM), lambda i: (0, 0))
    b_spec = pl.BlockSpec((1, DIM), lambda i: (0, 0))
    return pl.pallas_call(
        functools.partial(_dense_body, apply_relu),
        grid=grid,
        in_specs=[row_spec, part0_spec, part1_spec, part0_spec, part1_spec,
                  w_spec, w_spec, b_spec],
        out_specs=row_spec,
        out_shape=jax.ShapeDtypeStruct((NUM_NODES, DIM), jnp.float32),
    )(h, acc, acc, deg, deg, w_self, w_neigh, b.reshape(1, DIM))


def kernel(x, edge_index, W_self1, W_neigh1, b1, W_self2, W_neigh2, b2):
    src = edge_index[0]
    dst = edge_index[1]
    zrows = jnp.zeros((PAD_NODES, DIM), jnp.float32)
    ones = jnp.ones((CHUNK, DIM), jnp.float32)

    acc1, deg = _agg_with_deg(x, src, dst, zrows, ones)
    h1 = _dense(x, acc1, deg, W_self1, W_neigh1, b1, apply_relu=True)
    (acc2,) = _agg_no_deg(h1, src, dst, zrows)
    h2 = _dense(h1, acc2, deg, W_self2, W_neigh2, b2, apply_relu=False)
    return h2


# speedup vs baseline: 1.7668x; 1.3647x over previous
import functools

import jax
import jax.numpy as jnp
from jax import lax
from jax.experimental import pallas as pl
from jax.experimental.pallas import tpu as pltpu
from jax.experimental.pallas import tpu_sc as plsc

NUM_NODES = 10000
NUM_EDGES = 320000
DIM = 128
NCORES = 2
NSUB = 16
NWORK = NCORES * NSUB
CHUNK = 128
TOT_CHUNKS = NUM_EDGES // CHUNK
BASE_CHUNKS = TOT_CHUNKS // NWORK
EXTRA = TOT_CHUNKS - BASE_CHUNKS * NWORK
PAD_NODES = 10240
ROWS_PER_SUB = PAD_NODES // NSUB


def _agg_body(with_deg, *refs):
    if with_deg:
        (h_hbm, ei_hbm, zrows_hbm, ones_hbm,
         acc_out, deg_out,
         eidx0, eidx1, rows0, rows1, acc_sh, g0, g1) = refs
    else:
        (h_hbm, ei_hbm, zrows_hbm,
         acc_out,
         eidx0, eidx1, rows0, rows1, acc_sh, g0, g1) = refs

    cid = lax.axis_index("c")
    sid = lax.axis_index("s")
    wid = cid * NSUB + sid
    cstart = wid * BASE_CHUNKS + jnp.minimum(wid, EXTRA)
    nch = BASE_CHUNKS + jnp.where(wid < EXTRA, 1, 0)
    rbase = sid * ROWS_PER_SUB
    rslice = pl.ds(rbase, ROWS_PER_SUB)

    pltpu.sync_copy(zrows_hbm.at[rslice], acc_sh.at[rslice])
    plsc.subcore_barrier()

    def load_idx(c, eref):
        pltpu.sync_copy(ei_hbm.at[:, pl.ds((cstart + c) * CHUNK, CHUNK)], eref)

    def load_idx_async(c, eref, sem):
        pltpu.async_copy(ei_hbm.at[:, pl.ds((cstart + c) * CHUNK, CHUNK)],
                         eref, sem)

    def load_wait(eref, sem):
        pltpu.make_async_copy(ei_hbm.at[:, pl.ds(0, CHUNK)], eref, sem).wait()

    def gather(eref, rows, sem):
        pltpu.async_copy(h_hbm.at[eref.at[0]], rows, sem)

    def gather_wait(eref, rows, sem):
        pltpu.make_async_copy(h_hbm.at[eref.at[0]], rows, sem).wait()

    def scat(rows, eref):
        pltpu.sync_copy(rows, acc_sh.at[eref.at[1]], add=True)

    load_idx(0, eidx0)
    gather(eidx0, rows0, g0)

    @pl.loop(0, nch - 1, step=2)
    def _(i):
        load_idx(i + 1, eidx1)
        gather(eidx1, rows1, g1)
        gather_wait(eidx0, rows0, g0)
        scat(rows0, eidx0)

        @pl.when(i + 2 < nch)
        def _():
            load_idx(i + 2, eidx0)
            gather(eidx0, rows0, g0)

        gather_wait(eidx1, rows1, g1)
        scat(rows1, eidx1)

    @pl.when(nch % 2 == 1)
    def _():
        gather_wait(eidx0, rows0, g0)
        scat(rows0, eidx0)

    plsc.subcore_barrier()
    pltpu.sync_copy(acc_sh.at[rslice], acc_out.at[cid, rslice])

    if with_deg:
        pltpu.sync_copy(zrows_hbm.at[rslice], acc_sh.at[rslice])
        pltpu.sync_copy(ones_hbm, rows0)
        load_idx_async(0, eidx0, g0)
        plsc.subcore_barrier()

        @pl.loop(0, nch - 1, step=2)
        def _(i):
            load_idx_async(i + 1, eidx1, g1)
            load_wait(eidx0, g0)
            pltpu.sync_copy(rows0, acc_sh.at[eidx0.at[1]], add=True)

            @pl.when(i + 2 < nch)
            def _():
                load_idx_async(i + 2, eidx0, g0)

            load_wait(eidx1, g1)
            pltpu.sync_copy(rows0, acc_sh.at[eidx1.at[1]], add=True)

        @pl.when(nch % 2 == 1)
        def _():
            load_wait(eidx0, g0)
            pltpu.sync_copy(rows0, acc_sh.at[eidx0.at[1]], add=True)

        plsc.subcore_barrier()
        pltpu.sync_copy(acc_sh.at[rslice], deg_out.at[cid, rslice])


def _make_agg(with_deg):
    mesh = plsc.VectorSubcoreMesh(core_axis_name="c", subcore_axis_name="s")
    out_type = [jax.ShapeDtypeStruct((NCORES, PAD_NODES, DIM), jnp.float32)]
    scratch = [
        pltpu.VMEM((2, CHUNK), jnp.int32),
        pltpu.VMEM((2, CHUNK), jnp.int32),
        pltpu.VMEM((CHUNK, DIM), jnp.float32),
        pltpu.VMEM((CHUNK, DIM), jnp.float32),
        pltpu.VMEM_SHARED((PAD_NODES, DIM), jnp.float32),
        pltpu.SemaphoreType.DMA,
        pltpu.SemaphoreType.DMA,
    ]
    if with_deg:
        out_type.append(jax.ShapeDtypeStruct((NCORES, PAD_NODES, DIM),
                                             jnp.float32))
    return pl.kernel(functools.partial(_agg_body, with_deg),
                     out_type=out_type, mesh=mesh, scratch_types=scratch)


_agg_with_deg = _make_agg(True)
_agg_no_deg = _make_agg(False)

BN = 1000


def _dense_body(apply_relu, h_ref, a0_ref, a1_ref, d0_ref, d1_ref,
                ws_ref, wn_ref, b_ref, o_ref):
    deg = jnp.maximum(d0_ref[0, :, 0:1] + d1_ref[0, :, 0:1], 1.0)
    agg = (a0_ref[0] + a1_ref[0]) / deg
    out = (jnp.dot(h_ref[...], ws_ref[...],
                   preferred_element_type=jnp.float32,
                   precision=lax.Precision.HIGHEST)
           + jnp.dot(agg, wn_ref[...],
                     preferred_element_type=jnp.float32,
                     precision=lax.Precision.HIGHEST)
           + b_ref[...])
    if apply_relu:
        out = jnp.maximum(out, 0.0)
    o_ref[...] = out


def _dense(h, acc, deg, w_self, w_neigh, b, apply_relu):
    grid = (NUM_NODES // BN,)
    row_spec = pl.BlockSpec((BN, DIM), lambda i: (i, 0))
    part0_spec = pl.BlockSpec((1, BN, DIM), lambda i: (0, i, 0))
    part1_spec = pl.BlockSpec((1, BN, DIM), lambda i: (1, i, 0))
    w_spec = pl.BlockSpec((DIM, DIM), lambda i: (0, 0))
    b_spec = pl.BlockSpec((1, DIM), lambda i: (0, 0))
    return pl.pallas_call(
        functools.partial(_dense_body, apply_relu),
        grid=grid,
        in_specs=[row_spec, part0_spec, part1_spec, part0_spec, part1_spec,
                  w_spec, w_spec, b_spec],
        out_specs=row_spec,
        out_shape=jax.ShapeDtypeStruct((NUM_NODES, DIM), jnp.float32),
    )(h, acc, acc, deg, deg, w_self, w_neigh, b.reshape(1, DIM))


def kernel(x, edge_index, W_self1, W_neigh1, b1, W_self2, W_neigh2, b2):
    zrows = jnp.zeros((PAD_NODES, DIM), jnp.float32)
    ones = jnp.ones((CHUNK, DIM), jnp.float32)

    acc1, deg = _agg_with_deg(x, edge_index, zrows, ones)
    h1 = _dense(x, acc1, deg, W_self1, W_neigh1, b1, apply_relu=True)
    (acc2,) = _agg_no_deg(h1, edge_index, zrows)
    h2 = _dense(h1, acc2, deg, W_self2, W_neigh2, b2, apply_relu=False)
    return h2


# speedup vs baseline: 1.8129x; 1.0261x over previous
import functools

import jax
import jax.numpy as jnp
from jax import lax
from jax.experimental import pallas as pl
from jax.experimental.pallas import tpu as pltpu
from jax.experimental.pallas import tpu_sc as plsc

NUM_NODES = 10000
NUM_EDGES = 320000
DIM = 128
NCORES = 2
NSUB = 16
NWORK = NCORES * NSUB
CHUNK = 128
TOT_CHUNKS = NUM_EDGES // CHUNK
BASE_CHUNKS = TOT_CHUNKS // NWORK
EXTRA = TOT_CHUNKS - BASE_CHUNKS * NWORK
PAD_NODES = 10240
ROWS_PER_SUB = PAD_NODES // NSUB


def _agg_body(with_deg, *refs):
    if with_deg:
        (h_hbm, ei_hbm, zrows_hbm, ones_hbm,
         acc_out, deg_out,
         eidx0, eidx1, rows0, rows1, acc_sh, g0, g1) = refs
    else:
        (h_hbm, ei_hbm, zrows_hbm,
         acc_out,
         eidx0, eidx1, rows0, rows1, acc_sh, g0, g1) = refs

    cid = lax.axis_index("c")
    sid = lax.axis_index("s")
    wid = cid * NSUB + sid
    cstart = wid * BASE_CHUNKS + jnp.minimum(wid, EXTRA)
    nch = BASE_CHUNKS + jnp.where(wid < EXTRA, 1, 0)
    rbase = sid * ROWS_PER_SUB
    rslice = pl.ds(rbase, ROWS_PER_SUB)

    pltpu.sync_copy(zrows_hbm.at[rslice], acc_sh.at[rslice])
    plsc.subcore_barrier()

    def load_idx(c, eref):
        pltpu.sync_copy(ei_hbm.at[:, pl.ds((cstart + c) * CHUNK, CHUNK)], eref)

    def load_idx_async(c, eref, sem):
        pltpu.async_copy(ei_hbm.at[:, pl.ds((cstart + c) * CHUNK, CHUNK)],
                         eref, sem)

    def load_wait(eref, sem):
        pltpu.make_async_copy(ei_hbm.at[:, pl.ds(0, CHUNK)], eref, sem).wait()

    def gather(eref, rows, sem):
        pltpu.async_copy(h_hbm.at[eref.at[0]], rows, sem)

    def gather_wait(eref, rows, sem):
        pltpu.make_async_copy(h_hbm.at[eref.at[0]], rows, sem).wait()

    def scat(rows, eref):
        pltpu.sync_copy(rows, acc_sh.at[eref.at[1]], add=True)

    load_idx(0, eidx0)
    gather(eidx0, rows0, g0)

    @pl.loop(0, nch - 1, step=2)
    def _(i):
        load_idx(i + 1, eidx1)
        gather(eidx1, rows1, g1)
        gather_wait(eidx0, rows0, g0)
        scat(rows0, eidx0)

        @pl.when(i + 2 < nch)
        def _():
            load_idx(i + 2, eidx0)
            gather(eidx0, rows0, g0)

        gather_wait(eidx1, rows1, g1)
        scat(rows1, eidx1)

    @pl.when(nch % 2 == 1)
    def _():
        gather_wait(eidx0, rows0, g0)
        scat(rows0, eidx0)

    plsc.subcore_barrier()
    pltpu.sync_copy(acc_sh.at[rslice], acc_out.at[cid, rslice])

    if with_deg:
        pltpu.sync_copy(zrows_hbm.at[rslice], acc_sh.at[rslice])
        pltpu.sync_copy(ones_hbm, rows0)
        load_idx_async(0, eidx0, g0)
        plsc.subcore_barrier()

        @pl.loop(0, nch - 1, step=2)
        def _(i):
            load_idx_async(i + 1, eidx1, g1)
            load_wait(eidx0, g0)
            pltpu.sync_copy(rows0, acc_sh.at[eidx0.at[1]], add=True)

            @pl.when(i + 2 < nch)
            def _():
                load_idx_async(i + 2, eidx0, g0)

            load_wait(eidx1, g1)
            pltpu.sync_copy(rows0, acc_sh.at[eidx1.at[1]], add=True)

        @pl.when(nch % 2 == 1)
        def _():
            load_wait(eidx0, g0)
            pltpu.sync_copy(rows0, acc_sh.at[eidx0.at[1]], add=True)

        plsc.subcore_barrier()
        pltpu.sync_copy(acc_sh.at[rslice], deg_out.at[cid, rslice])


def _make_agg(with_deg):
    mesh = plsc.VectorSubcoreMesh(core_axis_name="c", subcore_axis_name="s")
    out_type = [jax.ShapeDtypeStruct((NCORES, PAD_NODES, DIM), jnp.float32)]
    scratch = [
        pltpu.VMEM((2, CHUNK), jnp.int32),
        pltpu.VMEM((2, CHUNK), jnp.int32),
        pltpu.VMEM((CHUNK, DIM), jnp.float32),
        pltpu.VMEM((CHUNK, DIM), jnp.float32),
        pltpu.VMEM_SHARED((PAD_NODES, DIM), jnp.float32),
        pltpu.SemaphoreType.DMA,
        pltpu.SemaphoreType.DMA,
    ]
    if with_deg:
        out_type.append(jax.ShapeDtypeStruct((NCORES, PAD_NODES, DIM),
                                             jnp.float32))
    return pl.kernel(functools.partial(_agg_body, with_deg),
                     out_type=out_type, mesh=mesh, scratch_types=scratch)


_agg_with_deg = _make_agg(True)
_agg_no_deg = _make_agg(False)

BN = 1000


def _self_body(h_ref, ws_ref, b_ref, o_ref):
    o_ref[...] = jnp.dot(h_ref[...], ws_ref[...],
                         preferred_element_type=jnp.float32,
                         precision=lax.Precision.HIGHEST) + b_ref[...]


def _self_dense(h, w_self, b):
    grid = (NUM_NODES // BN,)
    row_spec = pl.BlockSpec((BN, DIM), lambda i: (i, 0))
    w_spec = pl.BlockSpec((DIM, DIM), lambda i: (0, 0))
    b_spec = pl.BlockSpec((1, DIM), lambda i: (0, 0))
    return pl.pallas_call(
        _self_body,
        grid=grid,
        in_specs=[row_spec, w_spec, b_spec],
        out_specs=row_spec,
        out_shape=jax.ShapeDtypeStruct((NUM_NODES, DIM), jnp.float32),
    )(h, w_self, b.reshape(1, DIM))


def _rest_body(apply_relu, s_ref, a0_ref, a1_ref, d0_ref, d1_ref,
               wn_ref, o_ref):
    deg = jnp.maximum(d0_ref[0, :, 0:1] + d1_ref[0, :, 0:1], 1.0)
    agg = (a0_ref[0] + a1_ref[0]) / deg
    out = s_ref[...] + jnp.dot(agg, wn_ref[...],
                               preferred_element_type=jnp.float32,
                               precision=lax.Precision.HIGHEST)
    if apply_relu:
        out = jnp.maximum(out, 0.0)
    o_ref[...] = out


def _rest_dense(selfpart, acc, deg, w_neigh, apply_relu):
    grid = (NUM_NODES // BN,)
    row_spec = pl.BlockSpec((BN, DIM), lambda i: (i, 0))
    part0_spec = pl.BlockSpec((1, BN, DIM), lambda i: (0, i, 0))
    part1_spec = pl.BlockSpec((1, BN, DIM), lambda i: (1, i, 0))
    w_spec = pl.BlockSpec((DIM, DIM), lambda i: (0, 0))
    return pl.pallas_call(
        functools.partial(_rest_body, apply_relu),
        grid=grid,
        in_specs=[row_spec, part0_spec, part1_spec, part0_spec, part1_spec,
                  w_spec],
        out_specs=row_spec,
        out_shape=jax.ShapeDtypeStruct((NUM_NODES, DIM), jnp.float32),
    )(selfpart, acc, acc, deg, deg, w_neigh)


def kernel(x, edge_index, W_self1, W_neigh1, b1, W_self2, W_neigh2, b2):
    zrows = jnp.zeros((PAD_NODES, DIM), jnp.float32)
    ones = jnp.ones((CHUNK, DIM), jnp.float32)

    self1 = _self_dense(x, W_self1, b1)
    acc1, deg = _agg_with_deg(x, edge_index, zrows, ones)
    h1 = _rest_dense(self1, acc1, deg, W_neigh1, apply_relu=True)
    self2 = _self_dense(h1, W_self2, b2)
    (acc2,) = _agg_no_deg(h1, edge_index, zrows)
    h2 = _rest_dense(self2, acc2, deg, W_neigh2, apply_relu=False)
    return h2
